# baseline (speedup 1.0000x reference)
import functools

import jax
import jax.numpy as jnp
from jax import lax
from jax.experimental import pallas as pl
from jax.experimental.pallas import tpu as pltpu
from jax.experimental.pallas import tpu_sc as plsc

N = 10000
MP = 10240
F = 500
D = 16
E = 160000
NW = 32
CH = 128
NCH = 40
EPAD = NW * CH * NCH
RPT = MP // 16
DW = 8

_MESH = plsc.VectorSubcoreMesh(
    core_axis_name="c", subcore_axis_name="s", num_cores=2, num_subcores=16
)
_SC_PARAMS = pltpu.CompilerParams(use_tc_tiling_on_sc=False)


@functools.partial(
    pl.kernel,
    out_type=jax.ShapeDtypeStruct((2, MP, DW), jnp.float32),
    mesh=_MESH,
    scratch_types=[
        pltpu.VMEM((NCH, CH), jnp.int32),
        pltpu.VMEM((CH, DW), jnp.float32),
        pltpu.SemaphoreType.DMA,
        pltpu.MemorySpace.VMEM_SHARED((MP, DW), jnp.float32),
    ],
    compiler_params=_SC_PARAMS,
)
def _sc_deg(dst_hbm, zeros_hbm, ones_hbm, out_hbm, dst_v, ones_v, sem, acc_sh):
    cid = lax.axis_index("c")
    sid = lax.axis_index("s")
    wid = cid * 16 + sid
    rbase = sid * RPT
    pltpu.sync_copy(zeros_hbm.at[pl.ds(rbase, RPT)], acc_sh.at[pl.ds(rbase, RPT)])
    pltpu.sync_copy(ones_hbm, ones_v)
    pltpu.sync_copy(dst_hbm.at[wid], dst_v)
    plsc.subcore_barrier()

    def fire(j, carry):
        pltpu.async_copy(ones_v, acc_sh.at[dst_v.at[j]], sem, add=True)
        return carry

    def drain(j, carry):
        pltpu.make_async_copy(ones_v, acc_sh.at[dst_v.at[0]], sem).wait()
        return carry

    lax.fori_loop(0, NCH, fire, 0)
    lax.fori_loop(0, NCH, drain, 0)
    plsc.subcore_barrier()
    pltpu.sync_copy(acc_sh.at[pl.ds(rbase, RPT)], out_hbm.at[cid, pl.ds(rbase, RPT)])


@functools.partial(
    pl.kernel,
    out_type=jax.ShapeDtypeStruct((2, MP, D), jnp.float32),
    mesh=_MESH,
    scratch_types=[
        pltpu.VMEM((NCH, CH), jnp.int32),
        pltpu.VMEM((NCH, CH), jnp.int32),
        pltpu.VMEM((NCH * CH, D), jnp.float32),
        pltpu.SemaphoreType.DMA,
        pltpu.SemaphoreType.DMA,
        pltpu.MemorySpace.VMEM_SHARED((MP, D), jnp.float32),
    ],
    compiler_params=_SC_PARAMS,
)
def _sc_agg(g_hbm, src_hbm, dst_hbm, zeros_hbm, out_hbm,
            src_v, dst_v, big, sem_g, sem_s, acc_sh):
    cid = lax.axis_index("c")
    sid = lax.axis_index("s")
    wid = cid * 16 + sid
    rbase = sid * RPT
    pltpu.sync_copy(zeros_hbm.at[pl.ds(rbase, RPT)], acc_sh.at[pl.ds(rbase, RPT)])
    pltpu.sync_copy(src_hbm.at[wid], src_v)
    pltpu.sync_copy(dst_hbm.at[wid], dst_v)
    plsc.subcore_barrier()

    def fire_g(j, carry):
        pltpu.async_copy(g_hbm.at[src_v.at[j]], big.at[pl.ds(j * CH, CH)], sem_g)
        return carry

    def drain_g(j, carry):
        pltpu.make_async_copy(
            g_hbm.at[src_v.at[0]], big.at[pl.ds(0, CH)], sem_g).wait()
        return carry

    def fire_s(j, carry):
        pltpu.async_copy(
            big.at[pl.ds(j * CH, CH)], acc_sh.at[dst_v.at[j]], sem_s, add=True)
        return carry

    def drain_s(j, carry):
        pltpu.make_async_copy(
            big.at[pl.ds(0, CH)], acc_sh.at[dst_v.at[0]], sem_s).wait()
        return carry

    lax.fori_loop(0, NCH, fire_g, 0)
    lax.fori_loop(0, NCH, drain_g, 0)
    lax.fori_loop(0, NCH, fire_s, 0)
    lax.fori_loop(0, NCH, drain_s, 0)
    plsc.subcore_barrier()
    pltpu.sync_copy(acc_sh.at[pl.ds(rbase, RPT)], out_hbm.at[cid, pl.ds(rbase, RPT)])


def _mm_body(x_ref, w_ref, o_ref):
    o_ref[...] = jnp.dot(x_ref[...], w_ref[...], preferred_element_type=jnp.float32)


def _tc_matmul(x, w1):
    return pl.pallas_call(
        _mm_body,
        grid=(5,),
        in_specs=[
            pl.BlockSpec((2000, F), lambda i: (i, 0)),
            pl.BlockSpec((F, D), lambda i: (0, 0)),
        ],
        out_specs=pl.BlockSpec((2000, D), lambda i: (i, 0)),
        out_shape=jax.ShapeDtypeStruct((N, D), jnp.float32),
    )(x, w1)


def _prep_body(h1_ref, dp_ref, g1_ref, dis_ref):
    deg = dp_ref[0][:, 0:1] + dp_ref[1][:, 0:1] + 1.0
    dis = lax.rsqrt(deg)
    dis_ref[...] = dis
    g1_ref[0:N, :] = h1_ref[...] * dis[0:N, :]
    g1_ref[N:MP, :] = jnp.zeros((MP - N, D), jnp.float32)


def _tc_prep(h1, deg_part):
    return pl.pallas_call(
        _prep_body,
        out_shape=(
            jax.ShapeDtypeStruct((MP, D), jnp.float32),
            jax.ShapeDtypeStruct((MP, 1), jnp.float32),
        ),
    )(h1, deg_part)


def _mid_body(a_ref, g1_ref, dis_ref, b1_ref, g2_ref):
    acc = a_ref[0] + a_ref[1] + g1_ref[...]
    t = acc * dis_ref[...] + b1_ref[...]
    r = jnp.maximum(t, 0.0)
    g2 = r * dis_ref[...]
    row = lax.broadcasted_iota(jnp.int32, (MP, D), 0)
    g2_ref[...] = jnp.where(row < N, g2, 0.0)


def _tc_mid(acc1, g1, dis, b1):
    return pl.pallas_call(
        _mid_body,
        out_shape=jax.ShapeDtypeStruct((MP, D), jnp.float32),
    )(acc1, g1, dis, b1)


def _fin_body(a_ref, g2_ref, dis_ref, w2_ref, b2_ref, o_ref):
    acc = a_ref[0] + a_ref[1] + g2_ref[...]
    t = acc * dis_ref[...]
    h = jnp.dot(t, w2_ref[...], preferred_element_type=jnp.float32) + b2_ref[...]
    col = lax.broadcasted_iota(jnp.int32, (MP, D), 1)
    mask = col < 3
    m = jnp.where(mask, h, jnp.float32(-1e30))
    mx = jnp.max(m, axis=1, keepdims=True)
    e = jnp.where(mask, jnp.exp(m - mx), 0.0)
    lse = jnp.log(jnp.sum(e, axis=1, keepdims=True))
    o_ref[...] = m - mx - lse


def _tc_final(acc2, g2, dis, w2p, b2p):
    return pl.pallas_call(
        _fin_body,
        out_shape=jax.ShapeDtypeStruct((MP, D), jnp.float32),
    )(acc2, g2, dis, w2p, b2p)


@jax.jit
def kernel(x, edge, W1, b1, W2, b2):
    e32 = edge.astype(jnp.int32)
    pad = jnp.full((EPAD - E,), N, jnp.int32)
    src = jnp.concatenate([e32[0], pad]).reshape(NW, NCH, CH)
    dst = jnp.concatenate([e32[1], pad]).reshape(NW, NCH, CH)
    zeros_d = jnp.zeros((MP, D), jnp.float32)
    zeros_w = jnp.zeros((MP, DW), jnp.float32)
    ones_w = jnp.ones((CH, DW), jnp.float32)
    b1r = b1.reshape(1, D)
    w2p = jnp.pad(W2, ((0, 0), (0, D - 3)))
    b2p = jnp.pad(b2, (0, D - 3)).reshape(1, D)

    deg_part = _sc_deg(dst, zeros_w, ones_w)
    h1 = _tc_matmul(x, W1)
    g1, dis = _tc_prep(h1, deg_part)
    acc1 = _sc_agg(g1, src, dst, zeros_d)
    g2 = _tc_mid(acc1, g1, dis, b1r)
    acc2 = _sc_agg(g2, src, dst, zeros_d)
    out16 = _tc_final(acc2, g2, dis, w2p, b2p)
    return out16[:N, :3]


# speedup vs baseline: 1.2251x; 1.2251x over previous
import functools

import jax
import jax.numpy as jnp
from jax import lax
from jax.experimental import pallas as pl
from jax.experimental.pallas import tpu as pltpu
from jax.experimental.pallas import tpu_sc as plsc

N = 10000
MP = 10240
F = 500
D = 16
E = 160000
NW = 32
CH = 128
NCH = 40
EPAD = NW * CH * NCH
RPT = MP // 16
DW = 8

_MESH = plsc.VectorSubcoreMesh(
    core_axis_name="c", subcore_axis_name="s", num_cores=2, num_subcores=16
)
_SC_PARAMS = pltpu.CompilerParams(use_tc_tiling_on_sc=False)


@functools.partial(
    pl.kernel,
    out_type=jax.ShapeDtypeStruct((2, MP, DW), jnp.float32),
    mesh=_MESH,
    scratch_types=[
        pltpu.VMEM((NCH, CH), jnp.int32),
        pltpu.VMEM((CH, DW), jnp.float32),
        pltpu.SemaphoreType.DMA,
        pltpu.MemorySpace.VMEM_SHARED((MP, DW), jnp.float32),
    ],
    compiler_params=_SC_PARAMS,
)
def _sc_deg(dst_hbm, zeros_hbm, ones_hbm, out_hbm, dst_v, ones_v, sem, acc_sh):
    cid = lax.axis_index("c")
    sid = lax.axis_index("s")
    wid = cid * 16 + sid
    rbase = sid * RPT
    pltpu.sync_copy(zeros_hbm.at[pl.ds(rbase, RPT)], acc_sh.at[pl.ds(rbase, RPT)])
    pltpu.sync_copy(ones_hbm, ones_v)
    pltpu.sync_copy(dst_hbm.at[wid], dst_v)
    plsc.subcore_barrier()

    def fire(j, carry):
        pltpu.async_copy(ones_v, acc_sh.at[dst_v.at[j]], sem, add=True)
        return carry

    def drain(j, carry):
        pltpu.make_async_copy(ones_v, acc_sh.at[dst_v.at[0]], sem).wait()
        return carry

    lax.fori_loop(0, NCH, fire, 0)
    lax.fori_loop(0, NCH, drain, 0)
    plsc.subcore_barrier()
    pltpu.sync_copy(acc_sh.at[pl.ds(rbase, RPT)], out_hbm.at[cid, pl.ds(rbase, RPT)])


@functools.partial(
    pl.kernel,
    out_type=jax.ShapeDtypeStruct((2, MP, D), jnp.float32),
    mesh=_MESH,
    scratch_types=[
        pltpu.VMEM((NCH, CH), jnp.int32),
        pltpu.VMEM((NCH, CH), jnp.int32),
        pltpu.VMEM((NCH * CH, D), jnp.float32),
        pltpu.SemaphoreType.DMA,
        pltpu.SemaphoreType.DMA,
        pltpu.MemorySpace.VMEM_SHARED((MP, D), jnp.float32),
        pltpu.MemorySpace.VMEM_SHARED((MP, D), jnp.float32),
    ],
    compiler_params=_SC_PARAMS,
)
def _sc_agg(g_hbm, src_hbm, dst_hbm, zeros_hbm, out_hbm,
            src_v, dst_v, big, sem_g, sem_s, acc_sh, g_sh):
    cid = lax.axis_index("c")
    sid = lax.axis_index("s")
    wid = cid * 16 + sid
    rbase = sid * RPT
    pltpu.sync_copy(zeros_hbm.at[pl.ds(rbase, RPT)], acc_sh.at[pl.ds(rbase, RPT)])
    pltpu.sync_copy(g_hbm.at[pl.ds(rbase, RPT)], g_sh.at[pl.ds(rbase, RPT)])
    pltpu.sync_copy(src_hbm.at[wid], src_v)
    pltpu.sync_copy(dst_hbm.at[wid], dst_v)
    plsc.subcore_barrier()

    def fire_g(j, carry):
        pltpu.async_copy(g_sh.at[src_v.at[j]], big.at[pl.ds(j * CH, CH)], sem_g)
        return carry

    def drain_g(j, carry):
        pltpu.make_async_copy(
            g_sh.at[src_v.at[0]], big.at[pl.ds(0, CH)], sem_g).wait()
        return carry

    def fire_s(j, carry):
        pltpu.async_copy(
            big.at[pl.ds(j * CH, CH)], acc_sh.at[dst_v.at[j]], sem_s, add=True)
        return carry

    def drain_s(j, carry):
        pltpu.make_async_copy(
            big.at[pl.ds(0, CH)], acc_sh.at[dst_v.at[0]], sem_s).wait()
        return carry

    lax.fori_loop(0, NCH, fire_g, 0)
    lax.fori_loop(0, NCH, drain_g, 0)
    lax.fori_loop(0, NCH, fire_s, 0)
    lax.fori_loop(0, NCH, drain_s, 0)
    plsc.subcore_barrier()
    pltpu.sync_copy(acc_sh.at[pl.ds(rbase, RPT)], out_hbm.at[cid, pl.ds(rbase, RPT)])


def _mm_body(x_ref, w_ref, o_ref):
    o_ref[...] = jnp.dot(x_ref[...], w_ref[...], preferred_element_type=jnp.float32)


def _tc_matmul(x, w1):
    return pl.pallas_call(
        _mm_body,
        grid=(5,),
        in_specs=[
            pl.BlockSpec((2000, F), lambda i: (i, 0)),
            pl.BlockSpec((F, D), lambda i: (0, 0)),
        ],
        out_specs=pl.BlockSpec((2000, D), lambda i: (i, 0)),
        out_shape=jax.ShapeDtypeStruct((N, D), jnp.float32),
    )(x, w1)


def _prep_body(h1_ref, dp_ref, g1_ref, dis_ref):
    deg = dp_ref[0][:, 0:1] + dp_ref[1][:, 0:1] + 1.0
    dis = lax.rsqrt(deg)
    dis_ref[...] = dis
    g1_ref[0:N, :] = h1_ref[...] * dis[0:N, :]
    g1_ref[N:MP, :] = jnp.zeros((MP - N, D), jnp.float32)


def _tc_prep(h1, deg_part):
    return pl.pallas_call(
        _prep_body,
        out_shape=(
            jax.ShapeDtypeStruct((MP, D), jnp.float32),
            jax.ShapeDtypeStruct((MP, 1), jnp.float32),
        ),
    )(h1, deg_part)


def _mid_body(a_ref, g1_ref, dis_ref, b1_ref, g2_ref):
    acc = a_ref[0] + a_ref[1] + g1_ref[...]
    t = acc * dis_ref[...] + b1_ref[...]
    r = jnp.maximum(t, 0.0)
    g2 = r * dis_ref[...]
    row = lax.broadcasted_iota(jnp.int32, (MP, D), 0)
    g2_ref[...] = jnp.where(row < N, g2, 0.0)


def _tc_mid(acc1, g1, dis, b1):
    return pl.pallas_call(
        _mid_body,
        out_shape=jax.ShapeDtypeStruct((MP, D), jnp.float32),
    )(acc1, g1, dis, b1)


def _fin_body(a_ref, g2_ref, dis_ref, w2_ref, b2_ref, o_ref):
    acc = a_ref[0] + a_ref[1] + g2_ref[...]
    t = acc * dis_ref[...]
    h = jnp.dot(t, w2_ref[...], preferred_element_type=jnp.float32) + b2_ref[...]
    col = lax.broadcasted_iota(jnp.int32, (MP, D), 1)
    mask = col < 3
    m = jnp.where(mask, h, jnp.float32(-1e30))
    mx = jnp.max(m, axis=1, keepdims=True)
    e = jnp.where(mask, jnp.exp(m - mx), 0.0)
    lse = jnp.log(jnp.sum(e, axis=1, keepdims=True))
    o_ref[...] = m - mx - lse


def _tc_final(acc2, g2, dis, w2p, b2p):
    return pl.pallas_call(
        _fin_body,
        out_shape=jax.ShapeDtypeStruct((MP, D), jnp.float32),
    )(acc2, g2, dis, w2p, b2p)


@jax.jit
def kernel(x, edge, W1, b1, W2, b2):
    e32 = edge.astype(jnp.int32)
    pad = jnp.full((EPAD - E,), N, jnp.int32)
    src = jnp.concatenate([e32[0], pad]).reshape(NW, NCH, CH)
    dst = jnp.concatenate([e32[1], pad]).reshape(NW, NCH, CH)
    zeros_d = jnp.zeros((MP, D), jnp.float32)
    zeros_w = jnp.zeros((MP, DW), jnp.float32)
    ones_w = jnp.ones((CH, DW), jnp.float32)
    b1r = b1.reshape(1, D)
    w2p = jnp.pad(W2, ((0, 0), (0, D - 3)))
    b2p = jnp.pad(b2, (0, D - 3)).reshape(1, D)

    deg_part = _sc_deg(dst, zeros_w, ones_w)
    h1 = _tc_matmul(x, W1)
    g1, dis = _tc_prep(h1, deg_part)
    acc1 = _sc_agg(g1, src, dst, zeros_d)
    g2 = _tc_mid(acc1, g1, dis, b1r)
    acc2 = _sc_agg(g2, src, dst, zeros_d)
    out16 = _tc_final(acc2, g2, dis, w2p, b2p)
    return out16[:N, :3]


# speedup vs baseline: 1.2778x; 1.0431x over previous
import functools

import jax
import jax.numpy as jnp
from jax import lax
from jax.experimental import pallas as pl
from jax.experimental.pallas import tpu as pltpu
from jax.experimental.pallas import tpu_sc as plsc

N = 10000
MP = 10240
F = 500
D = 16
E = 160000
NW = 32
CH = 125
NCH = 40
RPT = MP // 16
DW = 8

_MESH = plsc.VectorSubcoreMesh(
    core_axis_name="c", subcore_axis_name="s", num_cores=2, num_subcores=16
)
_SC_PARAMS = pltpu.CompilerParams(use_tc_tiling_on_sc=False)


@functools.partial(
    pl.kernel,
    out_type=jax.ShapeDtypeStruct((2, MP, DW), jnp.float32),
    mesh=_MESH,
    scratch_types=[
        pltpu.VMEM((NCH, CH), jnp.int32),
        pltpu.VMEM((CH, DW), jnp.float32),
        pltpu.SemaphoreType.DMA,
        pltpu.MemorySpace.VMEM_SHARED((MP, DW), jnp.float32),
    ],
    compiler_params=_SC_PARAMS,
)
def _sc_deg(dst_hbm, zeros_hbm, ones_hbm, out_hbm, dst_v, ones_v, sem, acc_sh):
    cid = lax.axis_index("c")
    sid = lax.axis_index("s")
    wid = cid * 16 + sid
    rbase = sid * RPT
    pltpu.sync_copy(zeros_hbm.at[pl.ds(rbase, RPT)], acc_sh.at[pl.ds(rbase, RPT)])
    pltpu.sync_copy(ones_hbm, ones_v)
    pltpu.sync_copy(dst_hbm.at[wid], dst_v)
    plsc.subcore_barrier()

    def fire(j, carry):
        pltpu.async_copy(ones_v, acc_sh.at[dst_v.at[j]], sem, add=True)
        return carry

    def drain(j, carry):
        pltpu.make_async_copy(ones_v, acc_sh.at[dst_v.at[0]], sem).wait()
        return carry

    lax.fori_loop(0, NCH, fire, 0)
    lax.fori_loop(0, NCH, drain, 0)
    plsc.subcore_barrier()
    pltpu.sync_copy(acc_sh.at[pl.ds(rbase, RPT)], out_hbm.at[cid, pl.ds(rbase, RPT)])


@functools.partial(
    pl.kernel,
    out_type=jax.ShapeDtypeStruct((2, MP, D), jnp.float32),
    mesh=_MESH,
    scratch_types=[
        pltpu.VMEM((NCH, CH), jnp.int32),
        pltpu.VMEM((NCH, CH), jnp.int32),
        pltpu.VMEM((NCH * CH, D), jnp.float32),
        pltpu.SemaphoreType.DMA,
        pltpu.SemaphoreType.DMA,
        pltpu.MemorySpace.VMEM_SHARED((MP, D), jnp.float32),
        pltpu.MemorySpace.VMEM_SHARED((MP, D), jnp.float32),
    ],
    compiler_params=_SC_PARAMS,
)
def _sc_agg(g_hbm, src_hbm, dst_hbm, zeros_hbm, out_hbm,
            src_v, dst_v, big, sem_g, sem_s, acc_sh, g_sh):
    cid = lax.axis_index("c")
    sid = lax.axis_index("s")
    wid = cid * 16 + sid
    rbase = sid * RPT
    pltpu.sync_copy(zeros_hbm.at[pl.ds(rbase, RPT)], acc_sh.at[pl.ds(rbase, RPT)])
    pltpu.sync_copy(g_hbm.at[pl.ds(rbase, RPT)], g_sh.at[pl.ds(rbase, RPT)])
    pltpu.sync_copy(src_hbm.at[wid], src_v)
    pltpu.sync_copy(dst_hbm.at[wid], dst_v)
    plsc.subcore_barrier()

    def fire_g(j, carry):
        pltpu.async_copy(g_sh.at[src_v.at[j]], big.at[pl.ds(j * CH, CH)], sem_g)
        return carry

    def drain_g(j, carry):
        pltpu.make_async_copy(
            g_sh.at[src_v.at[0]], big.at[pl.ds(0, CH)], sem_g).wait()
        return carry

    def fire_s(j, carry):
        pltpu.async_copy(
            big.at[pl.ds(j * CH, CH)], acc_sh.at[dst_v.at[j]], sem_s, add=True)
        return carry

    def drain_s(j, carry):
        pltpu.make_async_copy(
            big.at[pl.ds(0, CH)], acc_sh.at[dst_v.at[0]], sem_s).wait()
        return carry

    lax.fori_loop(0, NCH, fire_g, 0)
    lax.fori_loop(0, NCH, drain_g, 0)
    lax.fori_loop(0, NCH, fire_s, 0)
    lax.fori_loop(0, NCH, drain_s, 0)
    plsc.subcore_barrier()
    pltpu.sync_copy(acc_sh.at[pl.ds(rbase, RPT)], out_hbm.at[cid, pl.ds(rbase, RPT)])


def _mm_body(x_ref, w_ref, o_ref):
    o_ref[...] = jnp.dot(x_ref[...], w_ref[...], preferred_element_type=jnp.float32)


def _tc_matmul(x, w1):
    return pl.pallas_call(
        _mm_body,
        grid=(5,),
        in_specs=[
            pl.BlockSpec((2000, F), lambda i: (i, 0)),
            pl.BlockSpec((F, D), lambda i: (0, 0)),
        ],
        out_specs=pl.BlockSpec((2000, D), lambda i: (i, 0)),
        out_shape=jax.ShapeDtypeStruct((N, D), jnp.float32),
    )(x, w1)


def _prep_body(h1_ref, dp_ref, g1_ref, dis_ref):
    deg = dp_ref[0][:, 0:1] + dp_ref[1][:, 0:1] + 1.0
    dis = lax.rsqrt(deg)
    dis_ref[...] = dis
    g1_ref[0:N, :] = h1_ref[...] * dis[0:N, :]
    g1_ref[N:MP, :] = jnp.zeros((MP - N, D), jnp.float32)


def _tc_prep(h1, deg_part):
    return pl.pallas_call(
        _prep_body,
        out_shape=(
            jax.ShapeDtypeStruct((MP, D), jnp.float32),
            jax.ShapeDtypeStruct((MP, 1), jnp.float32),
        ),
    )(h1, deg_part)


def _mid_body(a_ref, g1_ref, dis_ref, b1_ref, g2_ref):
    acc = a_ref[0] + a_ref[1] + g1_ref[...]
    t = acc * dis_ref[...] + b1_ref[...]
    r = jnp.maximum(t, 0.0)
    g2 = r * dis_ref[...]
    row = lax.broadcasted_iota(jnp.int32, (MP, D), 0)
    g2_ref[...] = jnp.where(row < N, g2, 0.0)


def _tc_mid(acc1, g1, dis, b1):
    return pl.pallas_call(
        _mid_body,
        out_shape=jax.ShapeDtypeStruct((MP, D), jnp.float32),
    )(acc1, g1, dis, b1)


def _fin_body(a_ref, g2_ref, dis_ref, w2_ref, b2_ref, o_ref):
    acc = a_ref[0] + a_ref[1] + g2_ref[...]
    t = acc * dis_ref[...]
    h = jnp.dot(t, w2_ref[...], preferred_element_type=jnp.float32) + b2_ref[...]
    col = lax.broadcasted_iota(jnp.int32, (MP, D), 1)
    mask = col < 3
    m = jnp.where(mask, h, jnp.float32(-1e30))
    mx = jnp.max(m, axis=1, keepdims=True)
    e = jnp.where(mask, jnp.exp(m - mx), 0.0)
    lse = jnp.log(jnp.sum(e, axis=1, keepdims=True))
    o_ref[...] = m - mx - lse


def _tc_final(acc2, g2, dis, w2p, b2p):
    return pl.pallas_call(
        _fin_body,
        out_shape=jax.ShapeDtypeStruct((MP, D), jnp.float32),
    )(acc2, g2, dis, w2p, b2p)


@jax.jit
def kernel(x, edge, W1, b1, W2, b2):
    e32 = edge.astype(jnp.int32).reshape(2, NW, NCH, CH)
    src = e32[0]
    dst = e32[1]
    zeros_d = jnp.zeros((MP, D), jnp.float32)
    zeros_w = jnp.zeros((MP, DW), jnp.float32)
    ones_w = jnp.ones((CH, DW), jnp.float32)
    b1r = b1.reshape(1, D)
    w2p = jnp.pad(W2, ((0, 0), (0, D - 3)))
    b2p = jnp.pad(b2, (0, D - 3)).reshape(1, D)

    deg_part = _sc_deg(dst, zeros_w, ones_w)
    h1 = _tc_matmul(x, W1)
    g1, dis = _tc_prep(h1, deg_part)
    acc1 = _sc_agg(g1, src, dst, zeros_d)
    g2 = _tc_mid(acc1, g1, dis, b1r)
    acc2 = _sc_agg(g2, src, dst, zeros_d)
    out16 = _tc_final(acc2, g2, dis, w2p, b2p)
    return out16[:N, :3]


# speedup vs baseline: 1.3546x; 1.0601x over previous
import functools

import jax
import jax.numpy as jnp
from jax import lax
from jax.experimental import pallas as pl
from jax.experimental.pallas import tpu as pltpu
from jax.experimental.pallas import tpu_sc as plsc

N = 10000
MP = 10240
F = 500
D = 16
E = 160000
NW = 32
CH = 125
NCH = 40
RPT = MP // 16
DW = 8

_MESH = plsc.VectorSubcoreMesh(
    core_axis_name="c", subcore_axis_name="s", num_cores=2, num_subcores=16
)
_SC_PARAMS = pltpu.CompilerParams(use_tc_tiling_on_sc=False)


@functools.partial(
    pl.kernel,
    out_type=jax.ShapeDtypeStruct((2, MP, DW), jnp.float32),
    mesh=_MESH,
    scratch_types=[
        pltpu.VMEM((NCH, CH), jnp.int32),
        pltpu.VMEM((CH, DW), jnp.float32),
        pltpu.SemaphoreType.DMA,
        pltpu.MemorySpace.VMEM_SHARED((MP, DW), jnp.float32),
    ],
    compiler_params=_SC_PARAMS,
)
def _sc_deg(edge_hbm, zeros_hbm, ones_hbm, out_hbm, dst_v, ones_v, sem, acc_sh):
    cid = lax.axis_index("c")
    sid = lax.axis_index("s")
    wid = cid * 16 + sid
    rbase = sid * RPT
    pltpu.sync_copy(zeros_hbm.at[pl.ds(rbase, RPT)], acc_sh.at[pl.ds(rbase, RPT)])
    pltpu.sync_copy(ones_hbm, ones_v)
    pltpu.sync_copy(edge_hbm.at[1, wid], dst_v)
    plsc.subcore_barrier()

    def fire(j, carry):
        pltpu.async_copy(ones_v, acc_sh.at[dst_v.at[j]], sem, add=True)
        return carry

    def drain(j, carry):
        pltpu.make_async_copy(ones_v, acc_sh.at[dst_v.at[0]], sem).wait()
        return carry

    lax.fori_loop(0, NCH, fire, 0)
    lax.fori_loop(0, NCH, drain, 0)
    plsc.subcore_barrier()
    pltpu.sync_copy(acc_sh.at[pl.ds(rbase, RPT)], out_hbm.at[cid, pl.ds(rbase, RPT)])


@functools.partial(
    pl.kernel,
    out_type=jax.ShapeDtypeStruct((2, MP, D), jnp.float32),
    mesh=_MESH,
    scratch_types=[
        pltpu.VMEM((NCH, CH), jnp.int32),
        pltpu.VMEM((NCH, CH), jnp.int32),
        pltpu.VMEM((NCH * CH, D), jnp.float32),
        pltpu.SemaphoreType.DMA,
        pltpu.SemaphoreType.DMA,
        pltpu.MemorySpace.VMEM_SHARED((MP, D), jnp.float32),
        pltpu.MemorySpace.VMEM_SHARED((MP, D), jnp.float32),
    ],
    compiler_params=_SC_PARAMS,
)
def _sc_agg(g_hbm, edge_hbm, zeros_hbm, out_hbm,
            src_v, dst_v, big, sem_g, sem_s, acc_sh, g_sh):
    cid = lax.axis_index("c")
    sid = lax.axis_index("s")
    wid = cid * 16 + sid
    rbase = sid * RPT
    pltpu.sync_copy(zeros_hbm.at[pl.ds(rbase, RPT)], acc_sh.at[pl.ds(rbase, RPT)])
    pltpu.sync_copy(g_hbm.at[pl.ds(rbase, RPT)], g_sh.at[pl.ds(rbase, RPT)])
    pltpu.sync_copy(edge_hbm.at[0, wid], src_v)
    pltpu.sync_copy(edge_hbm.at[1, wid], dst_v)
    plsc.subcore_barrier()

    def fire_g(j, carry):
        pltpu.async_copy(g_sh.at[src_v.at[j]], big.at[pl.ds(j * CH, CH)], sem_g)
        return carry

    def drain_g(j, carry):
        pltpu.make_async_copy(
            g_sh.at[src_v.at[0]], big.at[pl.ds(0, CH)], sem_g).wait()
        return carry

    def fire_s(j, carry):
        pltpu.async_copy(
            big.at[pl.ds(j * CH, CH)], acc_sh.at[dst_v.at[j]], sem_s, add=True)
        return carry

    def drain_s(j, carry):
        pltpu.make_async_copy(
            big.at[pl.ds(0, CH)], acc_sh.at[dst_v.at[0]], sem_s).wait()
        return carry

    lax.fori_loop(0, NCH, fire_g, 0)
    lax.fori_loop(0, NCH, drain_g, 0)
    lax.fori_loop(0, NCH, fire_s, 0)
    lax.fori_loop(0, NCH, drain_s, 0)
    plsc.subcore_barrier()
    pltpu.sync_copy(acc_sh.at[pl.ds(rbase, RPT)], out_hbm.at[cid, pl.ds(rbase, RPT)])


def _mm_body(x_ref, w_ref, o_ref):
    o_ref[...] = jnp.dot(x_ref[...], w_ref[...], preferred_element_type=jnp.float32)


def _tc_matmul(x, w1):
    return pl.pallas_call(
        _mm_body,
        grid=(5,),
        in_specs=[
            pl.BlockSpec((2000, F), lambda i: (i, 0)),
            pl.BlockSpec((F, D), lambda i: (0, 0)),
        ],
        out_specs=pl.BlockSpec((2000, D), lambda i: (i, 0)),
        out_shape=jax.ShapeDtypeStruct((N, D), jnp.float32),
    )(x, w1)


def _prep_body(h1_ref, dp_ref, g1_ref, dis_ref):
    deg = dp_ref[0][:, 0:1] + dp_ref[1][:, 0:1] + 1.0
    dis = lax.rsqrt(deg)
    dis_ref[...] = dis
    g1_ref[0:N, :] = h1_ref[...] * dis[0:N, :]
    g1_ref[N:MP, :] = jnp.zeros((MP - N, D), jnp.float32)


def _tc_prep(h1, deg_part):
    return pl.pallas_call(
        _prep_body,
        out_shape=(
            jax.ShapeDtypeStruct((MP, D), jnp.float32),
            jax.ShapeDtypeStruct((MP, 1), jnp.float32),
        ),
    )(h1, deg_part)


def _mid_body(a_ref, g1_ref, dis_ref, b1_ref, g2_ref):
    acc = a_ref[0] + a_ref[1] + g1_ref[...]
    t = acc * dis_ref[...] + b1_ref[...]
    r = jnp.maximum(t, 0.0)
    g2 = r * dis_ref[...]
    row = lax.broadcasted_iota(jnp.int32, (MP, D), 0)
    g2_ref[...] = jnp.where(row < N, g2, 0.0)


def _tc_mid(acc1, g1, dis, b1):
    return pl.pallas_call(
        _mid_body,
        out_shape=jax.ShapeDtypeStruct((MP, D), jnp.float32),
    )(acc1, g1, dis, b1)


def _fin_body(a_ref, g2_ref, dis_ref, w2_ref, b2_ref, o_ref):
    acc = a_ref[0] + a_ref[1] + g2_ref[...]
    t = acc * dis_ref[...]
    h = jnp.dot(t, w2_ref[...], preferred_element_type=jnp.float32) + b2_ref[...]
    col = lax.broadcasted_iota(jnp.int32, (MP, D), 1)
    mask = col < 3
    m = jnp.where(mask, h, jnp.float32(-1e30))
    mx = jnp.max(m, axis=1, keepdims=True)
    e = jnp.where(mask, jnp.exp(m - mx), 0.0)
    lse = jnp.log(jnp.sum(e, axis=1, keepdims=True))
    o_ref[...] = m - mx - lse


def _tc_final(acc2, g2, dis, w2p, b2p):
    return pl.pallas_call(
        _fin_body,
        out_shape=jax.ShapeDtypeStruct((MP, D), jnp.float32),
    )(acc2, g2, dis, w2p, b2p)


@jax.jit
def kernel(x, edge, W1, b1, W2, b2):
    e32 = edge.astype(jnp.int32).reshape(2, NW, NCH, CH)
    zeros_d = jnp.zeros((MP, D), jnp.float32)
    zeros_w = jnp.zeros((MP, DW), jnp.float32)
    ones_w = jnp.ones((CH, DW), jnp.float32)
    b1r = b1.reshape(1, D)
    w2p = jnp.pad(W2, ((0, 0), (0, D - 3)))
    b2p = jnp.pad(b2, (0, D - 3)).reshape(1, D)

    deg_part = _sc_deg(e32, zeros_w, ones_w)
    h1 = _tc_matmul(x, W1)
    g1, dis = _tc_prep(h1, deg_part)
    acc1 = _sc_agg(g1, e32, zeros_d)
    g2 = _tc_mid(acc1, g1, dis, b1r)
    acc2 = _sc_agg(g2, e32, zeros_d)
    out16 = _tc_final(acc2, g2, dis, w2p, b2p)
    return out16[:N, :3]


# speedup vs baseline: 2.0013x; 1.4774x over previous
import functools

import jax
import jax.numpy as jnp
from jax import lax
from jax.experimental import pallas as pl
from jax.experimental.pallas import tpu as pltpu
from jax.experimental.pallas import tpu_sc as plsc

N = 10000
MP = 10240
F = 500
D = 16
E = 160000
NW = 32
CH = 125
NCH = 40
RPT = MP // 16
DW = 16

_MESH = plsc.VectorSubcoreMesh(
    core_axis_name="c", subcore_axis_name="s", num_cores=2, num_subcores=16
)
_SC_PARAMS = pltpu.CompilerParams(use_tc_tiling_on_sc=False)


@functools.partial(
    pl.kernel,
    out_type=jax.ShapeDtypeStruct((2, MP, DW), jnp.float32),
    mesh=_MESH,
    scratch_types=[
        pltpu.VMEM((NCH, CH), jnp.int32),
        pltpu.VMEM((CH, DW), jnp.float32),
        pltpu.SemaphoreType.DMA,
        pltpu.MemorySpace.VMEM_SHARED((MP, DW), jnp.float32),
    ],
    compiler_params=_SC_PARAMS,
)
def _sc_deg(edge_hbm, zeros_hbm, ones_hbm, out_hbm, dst_v, ones_v, sem, acc_sh):
    cid = lax.axis_index("c")
    sid = lax.axis_index("s")
    wid = cid * 16 + sid
    rbase = sid * RPT
    pltpu.sync_copy(zeros_hbm.at[pl.ds(rbase, RPT)], acc_sh.at[pl.ds(rbase, RPT)])
    pltpu.sync_copy(ones_hbm, ones_v)
    pltpu.sync_copy(edge_hbm.at[1, wid], dst_v)
    plsc.subcore_barrier()

    def fire(j, carry):
        pltpu.async_copy(ones_v, acc_sh.at[dst_v.at[j]], sem, add=True)
        return carry

    def drain(j, carry):
        pltpu.make_async_copy(ones_v, acc_sh.at[dst_v.at[0]], sem).wait()
        return carry

    lax.fori_loop(0, NCH, fire, 0)
    lax.fori_loop(0, NCH, drain, 0)
    plsc.subcore_barrier()
    pltpu.sync_copy(acc_sh.at[pl.ds(rbase, RPT)], out_hbm.at[cid, pl.ds(rbase, RPT)])


@functools.partial(
    pl.kernel,
    out_type=jax.ShapeDtypeStruct((2, MP, D), jnp.float32),
    mesh=_MESH,
    scratch_types=[
        pltpu.VMEM((NCH, CH), jnp.int32),
        pltpu.VMEM((NCH, CH), jnp.int32),
        pltpu.VMEM((NCH * CH, D), jnp.float32),
        pltpu.SemaphoreType.DMA,
        pltpu.SemaphoreType.DMA,
        pltpu.MemorySpace.VMEM_SHARED((MP, D), jnp.float32),
        pltpu.MemorySpace.VMEM_SHARED((MP, D), jnp.float32),
    ],
    compiler_params=_SC_PARAMS,
)
def _sc_agg(g_hbm, edge_hbm, zeros_hbm, out_hbm,
            src_v, dst_v, big, sem_g, sem_s, acc_sh, g_sh):
    cid = lax.axis_index("c")
    sid = lax.axis_index("s")
    wid = cid * 16 + sid
    rbase = sid * RPT
    pltpu.sync_copy(zeros_hbm.at[pl.ds(rbase, RPT)], acc_sh.at[pl.ds(rbase, RPT)])
    pltpu.sync_copy(g_hbm.at[pl.ds(rbase, RPT)], g_sh.at[pl.ds(rbase, RPT)])
    pltpu.sync_copy(edge_hbm.at[0, wid], src_v)
    pltpu.sync_copy(edge_hbm.at[1, wid], dst_v)
    plsc.subcore_barrier()

    def fire_g(j, carry):
        pltpu.async_copy(g_sh.at[src_v.at[j]], big.at[pl.ds(j * CH, CH)], sem_g)
        return carry

    def drain_g(j, carry):
        pltpu.make_async_copy(
            g_sh.at[src_v.at[0]], big.at[pl.ds(0, CH)], sem_g).wait()
        return carry

    def fire_s(j, carry):
        pltpu.async_copy(
            big.at[pl.ds(j * CH, CH)], acc_sh.at[dst_v.at[j]], sem_s, add=True)
        return carry

    def drain_s(j, carry):
        pltpu.make_async_copy(
            big.at[pl.ds(0, CH)], acc_sh.at[dst_v.at[0]], sem_s).wait()
        return carry

    lax.fori_loop(0, NCH, fire_g, 0)
    lax.fori_loop(0, NCH, drain_g, 0)
    lax.fori_loop(0, NCH, fire_s, 0)
    lax.fori_loop(0, NCH, drain_s, 0)
    plsc.subcore_barrier()
    pltpu.sync_copy(acc_sh.at[pl.ds(rbase, RPT)], out_hbm.at[cid, pl.ds(rbase, RPT)])


def _mm_body(x_ref, w_ref, o_ref):
    o_ref[...] = jnp.dot(x_ref[...], w_ref[...], preferred_element_type=jnp.float32)


def _tc_matmul(x, w1):
    return pl.pallas_call(
        _mm_body,
        grid=(5,),
        in_specs=[
            pl.BlockSpec((2000, F), lambda i: (i, 0)),
            pl.BlockSpec((F, D), lambda i: (0, 0)),
        ],
        out_specs=pl.BlockSpec((2000, D), lambda i: (i, 0)),
        out_shape=jax.ShapeDtypeStruct((N, D), jnp.float32),
    )(x, w1)


NPK = N * D // 128
MPK = MP * D // 128


def _prep_body(h1_ref, dp_ref, g1_ref, dis_ref):
    deg = dp_ref[0] + dp_ref[1] + 1.0
    dis = lax.rsqrt(deg)
    dis_ref[...] = dis
    g1_ref[0:NPK, :] = h1_ref[...] * dis[0:NPK, :]
    g1_ref[NPK:MPK, :] = jnp.zeros((MPK - NPK, 128), jnp.float32)


def _tc_prep(h1p, deg_part):
    return pl.pallas_call(
        _prep_body,
        out_shape=(
            jax.ShapeDtypeStruct((MPK, 128), jnp.float32),
            jax.ShapeDtypeStruct((MPK, 128), jnp.float32),
        ),
    )(h1p, deg_part.reshape(2, MPK, 128))


def _mid_body(a_ref, g1_ref, dis_ref, b1_ref, g2_ref):
    acc = a_ref[0] + a_ref[1] + g1_ref[...]
    t = acc * dis_ref[...] + b1_ref[...]
    r = jnp.maximum(t, 0.0)
    g2 = r * dis_ref[...]
    row = lax.broadcasted_iota(jnp.int32, (MPK, 128), 0)
    g2_ref[...] = jnp.where(row < NPK, g2, 0.0)


def _tc_mid(acc1p, g1p, disp, b1t):
    return pl.pallas_call(
        _mid_body,
        out_shape=jax.ShapeDtypeStruct((MPK, 128), jnp.float32),
    )(acc1p, g1p, disp, b1t)


def _fin_body(a_ref, g2_ref, dis_ref, w2_ref, b2_ref, s_ref, o_ref):
    acc = a_ref[0] + a_ref[1] + g2_ref[...]
    tp = acc * dis_ref[...]
    h = jnp.dot(tp, w2_ref[...], preferred_element_type=jnp.float32) + b2_ref[...]
    col = lax.broadcasted_iota(jnp.int32, (MPK, 128), 1)
    mask = (col % 16) < 3
    hm = jnp.where(mask, h, jnp.float32(-1e30))
    mx = jnp.max(hm, axis=1, keepdims=True)
    e = jnp.where(mask, jnp.exp(h - mx), 0.0)
    s = jnp.dot(e, s_ref[...], preferred_element_type=jnp.float32)
    o_ref[...] = h - mx - jnp.log(s)


def _tc_final(acc2p, g2p, disp, w2bd, b2t, ssum):
    return pl.pallas_call(
        _fin_body,
        out_shape=jax.ShapeDtypeStruct((MPK, 128), jnp.float32),
    )(acc2p, g2p, disp, w2bd, b2t, ssum)


@jax.jit
def kernel(x, edge, W1, b1, W2, b2):
    e32 = edge.astype(jnp.int32).reshape(2, NW, NCH, CH)
    zeros_d = jnp.zeros((MP, D), jnp.float32)
    ones_w = jnp.ones((CH, DW), jnp.float32)
    b1t = jnp.tile(b1, 8).reshape(1, 128)
    w2p = jnp.pad(W2, ((0, 0), (0, D - 3)))
    w2bd = jnp.kron(jnp.eye(8, dtype=jnp.float32), w2p)
    b2t = jnp.tile(jnp.pad(b2, (0, D - 3)), 8).reshape(1, 128)
    sblk = jnp.zeros((D, D), jnp.float32).at[:3, :3].set(1.0)
    ssum = jnp.kron(jnp.eye(8, dtype=jnp.float32), sblk)

    deg_part = _sc_deg(e32, zeros_d, ones_w)
    h1 = _tc_matmul(x, W1)
    g1p, disp = _tc_prep(h1.reshape(NPK, 128), deg_part)
    acc1 = _sc_agg(g1p.reshape(MP, D), e32, zeros_d)
    g2p = _tc_mid(acc1.reshape(2, MPK, 128), g1p, disp, b1t)
    acc2 = _sc_agg(g2p.reshape(MP, D), e32, zeros_d)
    outp = _tc_final(acc2.reshape(2, MPK, 128), g2p, disp, w2bd, b2t, ssum)
    return outp.reshape(MP, D)[:N, :3]


# speedup vs baseline: 2.4197x; 1.2091x over previous
import functools

import jax
import jax.numpy as jnp
from jax import lax
from jax.experimental import pallas as pl
from jax.experimental.pallas import tpu as pltpu
from jax.experimental.pallas import tpu_sc as plsc

N = 10000
MP = 10240
F = 500
D = 16
E = 160000
NW = 32
CH = 125
NCH = 40
RPT = MP // 16
DW = 16

_MESH = plsc.VectorSubcoreMesh(
    core_axis_name="c", subcore_axis_name="s", num_cores=2, num_subcores=16
)
_SC_PARAMS = pltpu.CompilerParams(use_tc_tiling_on_sc=False)


@functools.partial(
    pl.kernel,
    out_type=jax.ShapeDtypeStruct((2, MP, DW), jnp.float32),
    mesh=_MESH,
    scratch_types=[
        pltpu.VMEM((NCH, CH), jnp.int32),
        pltpu.VMEM((CH, DW), jnp.float32),
        pltpu.SemaphoreType.DMA,
        pltpu.MemorySpace.VMEM_SHARED((MP, DW), jnp.float32),
    ],
    compiler_params=_SC_PARAMS,
)
def _sc_deg(edge_hbm, zeros_hbm, ones_hbm, out_hbm, dst_v, ones_v, sem, acc_sh):
    cid = lax.axis_index("c")
    sid = lax.axis_index("s")
    wid = cid * 16 + sid
    rbase = sid * RPT
    pltpu.sync_copy(zeros_hbm.at[pl.ds(rbase, RPT)], acc_sh.at[pl.ds(rbase, RPT)])
    pltpu.sync_copy(ones_hbm, ones_v)
    pltpu.sync_copy(edge_hbm.at[1, wid], dst_v)
    plsc.subcore_barrier()

    def fire(j, carry):
        pltpu.async_copy(ones_v, acc_sh.at[dst_v.at[j]], sem, add=True)
        return carry

    def drain(j, carry):
        pltpu.make_async_copy(ones_v, acc_sh.at[dst_v.at[0]], sem).wait()
        return carry

    lax.fori_loop(0, NCH, fire, 0)
    lax.fori_loop(0, NCH, drain, 0)
    plsc.subcore_barrier()
    pltpu.sync_copy(acc_sh.at[pl.ds(rbase, RPT)], out_hbm.at[cid, pl.ds(rbase, RPT)])


@functools.partial(
    pl.kernel,
    out_type=jax.ShapeDtypeStruct((2, MP, D), jnp.float32),
    mesh=_MESH,
    scratch_types=[
        pltpu.VMEM((NCH, CH), jnp.int32),
        pltpu.VMEM((NCH, CH), jnp.int32),
        pltpu.VMEM((NCH * CH, D), jnp.float32),
        pltpu.SemaphoreType.DMA,
        pltpu.SemaphoreType.DMA,
        pltpu.MemorySpace.VMEM_SHARED((MP, D), jnp.float32),
        pltpu.MemorySpace.VMEM_SHARED((MP, D), jnp.float32),
    ],
    compiler_params=_SC_PARAMS,
)
def _sc_agg(g_hbm, edge_hbm, zeros_hbm, out_hbm,
            src_v, dst_v, big, sem_g, sem_s, acc_sh, g_sh):
    cid = lax.axis_index("c")
    sid = lax.axis_index("s")
    wid = cid * 16 + sid
    rbase = sid * RPT
    pltpu.sync_copy(zeros_hbm.at[pl.ds(rbase, RPT)], acc_sh.at[pl.ds(rbase, RPT)])
    pltpu.sync_copy(g_hbm.at[pl.ds(rbase, RPT)], g_sh.at[pl.ds(rbase, RPT)])
    pltpu.sync_copy(edge_hbm.at[0, wid], src_v)
    pltpu.sync_copy(edge_hbm.at[1, wid], dst_v)
    plsc.subcore_barrier()

    def fire_g(j, carry):
        pltpu.async_copy(g_sh.at[src_v.at[j]], big.at[pl.ds(j * CH, CH)], sem_g)
        return carry

    def drain_g(j, carry):
        pltpu.make_async_copy(
            g_sh.at[src_v.at[0]], big.at[pl.ds(0, CH)], sem_g).wait()
        return carry

    def fire_s(j, carry):
        pltpu.async_copy(
            big.at[pl.ds(j * CH, CH)], acc_sh.at[dst_v.at[j]], sem_s, add=True)
        return carry

    def drain_s(j, carry):
        pltpu.make_async_copy(
            big.at[pl.ds(0, CH)], acc_sh.at[dst_v.at[0]], sem_s).wait()
        return carry

    lax.fori_loop(0, NCH, fire_g, 0)
    lax.fori_loop(0, NCH, drain_g, 0)
    lax.fori_loop(0, NCH, fire_s, 0)
    lax.fori_loop(0, NCH, drain_s, 0)
    plsc.subcore_barrier()
    pltpu.sync_copy(acc_sh.at[pl.ds(rbase, RPT)], out_hbm.at[cid, pl.ds(rbase, RPT)])


def _mm_body(xt_ref, w_ref, o_ref):
    o_ref[...] = lax.dot_general(
        xt_ref[...], w_ref[...],
        dimension_numbers=(((0,), (0,)), ((), ())),
        preferred_element_type=jnp.float32,
    )


def _tc_matmul(xt, w1):
    return pl.pallas_call(
        _mm_body,
        grid=(5,),
        in_specs=[
            pl.BlockSpec((F, 2048), lambda i: (0, i)),
            pl.BlockSpec((F, D), lambda i: (0, 0)),
        ],
        out_specs=pl.BlockSpec((2048, D), lambda i: (i, 0)),
        out_shape=jax.ShapeDtypeStruct((MP, D), jnp.float32),
    )(xt, w1)


NPK = N * D // 128
MPK = MP * D // 128


def _prep_body(h1_ref, dp_ref, g1_ref, dis_ref):
    deg = dp_ref[0] + dp_ref[1] + 1.0
    dis = lax.rsqrt(deg)
    dis_ref[...] = dis
    g1_ref[0:NPK, :] = h1_ref[0:NPK, :] * dis[0:NPK, :]
    g1_ref[NPK:MPK, :] = jnp.zeros((MPK - NPK, 128), jnp.float32)


def _tc_prep(h1p, deg_part):
    return pl.pallas_call(
        _prep_body,
        out_shape=(
            jax.ShapeDtypeStruct((MPK, 128), jnp.float32),
            jax.ShapeDtypeStruct((MPK, 128), jnp.float32),
        ),
    )(h1p, deg_part.reshape(2, MPK, 128))


def _mid_body(a_ref, g1_ref, dis_ref, b1_ref, g2_ref):
    acc = a_ref[0] + a_ref[1] + g1_ref[...]
    t = acc * dis_ref[...] + b1_ref[...]
    r = jnp.maximum(t, 0.0)
    g2 = r * dis_ref[...]
    row = lax.broadcasted_iota(jnp.int32, (MPK, 128), 0)
    g2_ref[...] = jnp.where(row < NPK, g2, 0.0)


def _tc_mid(acc1p, g1p, disp, b1t):
    return pl.pallas_call(
        _mid_body,
        out_shape=jax.ShapeDtypeStruct((MPK, 128), jnp.float32),
    )(acc1p, g1p, disp, b1t)


def _fin_body(a_ref, g2_ref, dis_ref, w2_ref, b2_ref, s_ref, o_ref):
    acc = a_ref[0] + a_ref[1] + g2_ref[...]
    tp = acc * dis_ref[...]
    h = jnp.dot(tp, w2_ref[...], preferred_element_type=jnp.float32) + b2_ref[...]
    col = lax.broadcasted_iota(jnp.int32, (MPK, 128), 1)
    mask = (col % 16) < 3
    hm = jnp.where(mask, h, jnp.float32(-1e30))
    mx = jnp.max(hm, axis=1, keepdims=True)
    e = jnp.where(mask, jnp.exp(h - mx), 0.0)
    s = jnp.dot(e, s_ref[...], preferred_element_type=jnp.float32)
    o_ref[...] = h - mx - jnp.log(s)


def _tc_final(acc2p, g2p, disp, w2bd, b2t, ssum):
    return pl.pallas_call(
        _fin_body,
        out_shape=jax.ShapeDtypeStruct((MPK, 128), jnp.float32),
    )(acc2p, g2p, disp, w2bd, b2t, ssum)


@jax.jit
def kernel(x, edge, W1, b1, W2, b2):
    e32 = edge.astype(jnp.int32).reshape(2, NW, NCH, CH)
    zeros_d = jnp.zeros((MP, D), jnp.float32)
    ones_w = jnp.ones((CH, DW), jnp.float32)
    b1t = jnp.tile(b1, 8).reshape(1, 128)
    w2p = jnp.pad(W2, ((0, 0), (0, D - 3)))
    w2bd = jnp.kron(jnp.eye(8, dtype=jnp.float32), w2p)
    b2t = jnp.tile(jnp.pad(b2, (0, D - 3)), 8).reshape(1, 128)
    sblk = jnp.zeros((D, D), jnp.float32).at[:3, :3].set(1.0)
    ssum = jnp.kron(jnp.eye(8, dtype=jnp.float32), sblk)

    deg_part = _sc_deg(e32, zeros_d, ones_w)
    h1 = _tc_matmul(x.T, W1)
    g1p, disp = _tc_prep(h1.reshape(MPK, 128), deg_part)
    acc1 = _sc_agg(g1p.reshape(MP, D), e32, zeros_d)
    g2p = _tc_mid(acc1.reshape(2, MPK, 128), g1p, disp, b1t)
    acc2 = _sc_agg(g2p.reshape(MP, D), e32, zeros_d)
    outp = _tc_final(acc2.reshape(2, MPK, 128), g2p, disp, w2bd, b2t, ssum)
    return outp.reshape(MP, D)[:N, :3]
